# baseline (speedup 1.0000x reference)
import functools

import jax
import jax.numpy as jnp
from jax.experimental import pallas as pl
from jax.experimental.pallas import tpu as pltpu

_EPS = 1e-12
_LANES = 128
_BLK = 136


def _loss_kernel(x0, x1, x2, x3, t0, t1, t2, t3, t4, out_ref, acc_ref,
                 *, n_blocks):
    i = pl.program_id(0)

    @pl.when(i == 0)
    def _init():
        acc_ref[0] = 0.0
        acc_ref[1] = 0.0

    obj = t4[...] > 0.0

    def bce(x_ref, t_ref):
        p = jnp.clip(x_ref[...], _EPS, 1.0 - _EPS)
        t = t_ref[...]
        return -(t * jnp.log(p) + (1.0 - t) * jnp.log(1.0 - p))

    elem = bce(x0, t0) + bce(x1, t1) + bce(x2, t2) + bce(x3, t3)
    acc_ref[0] += jnp.sum(jnp.where(obj, elem, 0.0))
    acc_ref[1] += jnp.sum(jnp.where(obj, 1.0, 0.0))

    @pl.when(i == n_blocks - 1)
    def _finalize():
        denom = jnp.maximum(acc_ref[1] * 2.0, 1.0)
        out_ref[...] = jnp.full((1, 1), acc_ref[0] / denom, jnp.float32)


def kernel(x, target):
    b, n, c = x.shape
    rows = b * n
    n_blocks = pl.cdiv(rows, _BLK * _LANES)
    padded = n_blocks * _BLK * _LANES
    pad = jnp.zeros((padded - rows,), jnp.float32)

    def plane(a, ch):
        return jnp.concatenate([a[:, :, ch].reshape(-1), pad]).reshape(
            n_blocks * _BLK, _LANES)

    planes = [plane(x, ch) for ch in range(4)]
    planes += [plane(target, ch) for ch in range(5)]

    spec = pl.BlockSpec((_BLK, _LANES), lambda i: (i, 0))
    out = pl.pallas_call(
        functools.partial(_loss_kernel, n_blocks=n_blocks),
        grid=(n_blocks,),
        in_specs=[spec] * 9,
        out_specs=pl.BlockSpec((1, 1), lambda i: (0, 0)),
        out_shape=jax.ShapeDtypeStruct((1, 1), jnp.float32),
        scratch_shapes=[pltpu.SMEM((2,), jnp.float32)],
    )(*planes)
    return out[0, 0]
